# baseline (speedup 1.0000x reference)
import functools

import jax
import jax.numpy as jnp
from jax import lax
from jax.experimental import pallas as pl
from jax.experimental.pallas import tpu as pltpu
from jax.experimental.pallas import tpu_sc as plsc

N = 10000
E = 160000
G = 16
H = 128
NH = 8
DH = 16

NPAD = 10240
BN = 2048
NBLK = NPAD // BN
EPAD = 163840
EB = 2048
NEBLK = EPAD // EB
NW = 32
EPW = EPAD // NW
SCB = 128
NCHUNK = EPW // SCB
NGRP = SCB // 16
NSUB = 8
SUBR = SCB // NSUB
ZROWS = NPAD // 16


def _gelu(x):
    return 0.5 * x * (1.0 + lax.erf(x * 0.7071067811865476))


def _ln_rows(x, g, b):
    m = jnp.mean(x, axis=-1, keepdims=True)
    v = jnp.mean((x - m) ** 2, axis=-1, keepdims=True)
    return (x - m) * jax.lax.rsqrt(v + 1e-5) * g + b


def _embed_body(x_ref, pw, pb, pg, pbe, lw, lb, lg, lbe, out_ref):
    i = pl.program_id(0)
    xb = x_ref[...]
    rows = i * BN + lax.broadcasted_iota(jnp.int32, (BN, 1), 0)
    is_prot = rows < 512
    lin_p = xb @ pw[...] + pb[...]
    lin_l = xb @ lw[...] + lb[...]
    lin = jnp.where(is_prot, lin_p, lin_l)
    g = jnp.where(is_prot, pg[...], lg[...])
    b = jnp.where(is_prot, pbe[...], lbe[...])
    out_ref[...] = _gelu(_ln_rows(lin, g, b))


def _embed(xp, pw, pb, pg, pbe, lw, lb, lg, lbe):
    full = pl.BlockSpec((1, H), lambda i: (0, 0))
    wspec = pl.BlockSpec((16, H), lambda i: (0, 0))
    return pl.pallas_call(
        _embed_body,
        grid=(NBLK,),
        in_specs=[pl.BlockSpec((BN, 16), lambda i: (i, 0)),
                  wspec, full, full, full, wspec, full, full, full],
        out_specs=pl.BlockSpec((BN, H), lambda i: (i, 0)),
        out_shape=jax.ShapeDtypeStruct((NPAD, H), jnp.float32),
    )(xp, pw, pb, pg, pbe, lw, lb, lg, lbe)


def _qkv_body(h_ref, ng, nb, qw1, qb1, qw2, qb2, kw1, kb1, kw2, kb2,
              vw1, vb1, vw2, vb2, cs, q_ref, k_ref, v_ref):
    hn = _ln_rows(h_ref[...], ng[...], nb[...])
    q = _gelu(hn @ qw1[...] + qb1[...]) @ qw2[...] + qb2[...]
    q_ref[...] = q.astype(jnp.bfloat16)
    k = (_gelu(hn @ kw1[...] + kb1[...]) @ kw2[...] + kb2[...]) * cs[...]
    k_ref[...] = k.astype(jnp.bfloat16)
    v_ref[...] = _gelu(hn @ vw1[...] + vb1[...]) @ vw2[...] + vb2[...]


def _qkv(h, ng, nb, qp, kp, vp, cs):
    b1 = pl.BlockSpec((1, 2 * H), lambda i: (0, 0))
    b2 = pl.BlockSpec((1, H), lambda i: (0, 0))
    w1 = pl.BlockSpec((H, 2 * H), lambda i: (0, 0))
    w2 = pl.BlockSpec((2 * H, H), lambda i: (0, 0))
    nspec = pl.BlockSpec((BN, H), lambda i: (i, 0))
    return pl.pallas_call(
        _qkv_body,
        grid=(NBLK,),
        in_specs=[nspec, b2, b2, w1, b1, w2, b2, w1, b1, w2, b2,
                  w1, b1, w2, b2, b2],
        out_specs=[nspec, nspec, nspec],
        out_shape=[jax.ShapeDtypeStruct((NPAD, H), jnp.bfloat16),
                   jax.ShapeDtypeStruct((NPAD, H), jnp.bfloat16),
                   jax.ShapeDtypeStruct((NPAD, H), jnp.float32)],
    )(h, ng, nb,
      qp["W1"], qp["b1"].reshape(1, -1), qp["W2"], qp["b2"].reshape(1, -1),
      kp["W1"], kp["b1"].reshape(1, -1), kp["W2"], kp["b2"].reshape(1, -1),
      vp["W1"], vp["b1"].reshape(1, -1), vp["W2"], vp["b2"].reshape(1, -1),
      cs)


def _spc_body(ea_ref, w1, b1, w2, b2, o0, o1, o2, o3):
    hid = jax.nn.relu(ea_ref[...] @ w1[...] + b1[...])
    for l, o in enumerate((o0, o1, o2, o3)):
        wl = w2[pl.ds(l * 64, 64), :]
        bl = b2[:, pl.ds(l * 16, 16)]
        o[...] = hid[:, l * 64:(l + 1) * 64] @ wl + bl


def _spc(eap, w1, b1, w2, b2):
    espec = pl.BlockSpec((EB, 8), lambda i: (i, 0))
    ospec = pl.BlockSpec((EB, 16), lambda i: (i, 0))
    return pl.pallas_call(
        _spc_body,
        grid=(NEBLK,),
        in_specs=[espec,
                  pl.BlockSpec((8, 256), lambda i: (0, 0)),
                  pl.BlockSpec((1, 256), lambda i: (0, 0)),
                  pl.BlockSpec((256, 16), lambda i: (0, 0)),
                  pl.BlockSpec((1, 64), lambda i: (0, 0))],
        out_specs=[ospec] * 4,
        out_shape=[jax.ShapeDtypeStruct((EPAD, 16), jnp.float32)] * 4,
    )(eap, w1, b1, w2, b2)


def _edge_body(q_hbm, k_hbm, spc_hbm, src_hbm, dst_hbm, s2_hbm,
               srcs, dsts, qrows0, qrows1, krows0, krows1, spcv0, spcv1,
               wbuf0, wbuf1, zbuf,
               semq0, semq1, semk0, semk1, semp0, semp1, sems0, sems1,
               semt0, s_sh, q_sh):
    cid = lax.axis_index("c")
    sid = lax.axis_index("s")
    wid = cid * 16 + sid
    ebase = wid * EPW

    rows = pl.ds(sid * ZROWS, ZROWS)
    st_q = pltpu.async_copy(q_hbm.at[rows, :], q_sh.at[rows, :], semt0)

    qrows = (qrows0, qrows1)
    krows = (krows0, krows1)
    spcv = (spcv0, spcv1)
    wbuf = (wbuf0, wbuf1)
    semq = (semq0, semq1)
    semk = (semk0, semk1)
    semp = (semp0, semp1)
    sems = (sems0, sems1)

    pltpu.sync_copy(src_hbm.at[wid], srcs)
    pltpu.sync_copy(dst_hbm.at[wid], dsts)

    @pl.loop(0, SCB)
    def _zw(r):
        z = jnp.zeros((16,), jnp.float32)
        wbuf0[r, :] = z
        wbuf1[r, :] = z
        zbuf[r, :] = z

    for j in range(ZROWS // SCB):
        pltpu.sync_copy(zbuf, s_sh.at[pl.ds(sid * ZROWS + j * SCB, SCB), :])
    st_q.wait()
    plsc.subcore_barrier()

    lane = lax.broadcasted_iota(jnp.int32, (16,), 0)

    def _issue(ci, b):
        goff = pl.multiple_of(ebase + ci * SCB, 8)
        for s in range(NSUB):
            rs = pl.ds(s * SUBR, SUBR)
            pltpu.async_copy(q_sh.at[srcs.at[ci, rs]],
                             qrows[b].at[rs, :], semq[b])
            pltpu.async_copy(k_hbm.at[dsts.at[ci, rs]],
                             krows[b].at[rs, :], semk[b])
        pltpu.async_copy(spc_hbm.at[pl.ds(goff, SCB), :], spcv[b], semp[b])

    def _wait_gather(b):
        for s in range(NSUB):
            rs = pl.ds(s * SUBR, SUBR)
            pltpu.make_async_copy(q_sh.at[pl.ds(0, SUBR), :],
                                  qrows[b].at[rs, :], semq[b]).wait()
            pltpu.make_async_copy(k_hbm.at[pl.ds(0, SUBR), :],
                                  krows[b].at[rs, :], semk[b]).wait()
        pltpu.make_async_copy(spc_hbm.at[pl.ds(0, SCB), :], spcv[b], semp[b]).wait()

    def _wait_scatter(b):
        pltpu.make_async_copy(wbuf[b], s_sh.at[pl.ds(0, SCB), :], sems[b]).wait()

    lane7 = lane == 7
    lane15 = lane == 15
    lane_hi = lane >= 8

    def _compute(b):
        qb, kb, pb, wb = qrows[b], krows[b], spcv[b], wbuf[b]

        @pl.loop(0, SCB)
        def _edge(e):
            erep = jnp.full((16,), e, jnp.int32)
            for j in range(NH // 2):
                qp = qb[e, pl.ds(j * 2 * DH, 2 * DH)]
                kp = kb[e, pl.ds(j * 2 * DH, 2 * DH)]
                qa, qo = plsc.unpack(qp, format=plsc.PackFormat.INTERLEAVED)
                ka, ko = plsc.unpack(kp, format=plsc.PackFormat.INTERLEAVED)
                prod = qa * ka + qo * ko
                s_lo = plsc.cumsum(prod)
                s_hi = plsc.cumsum(jnp.where(lane_hi, prod, 0.0))
                plsc.store_scatter(
                    wb, [erep, jnp.full((16,), 2 * j, jnp.int32)], s_lo,
                    mask=lane7)
                plsc.store_scatter(
                    wb, [erep, jnp.full((16,), 2 * j + 1, jnp.int32)], s_hi,
                    mask=lane15)

        @pl.loop(0, SCB)
        def _edge2(e):
            logit = wb[e, :] + pb[e, :]
            sig = 1.0 / (1.0 + jnp.exp(-logit))
            wb[e, :] = jnp.exp(sig)

    def _half(ci, b, first):
        if isinstance(ci, int):
            if ci + 1 < NCHUNK:
                _issue(ci + 1, 1 - b)
        else:
            @pl.when(ci + 1 < NCHUNK)
            def _():
                _issue(ci + 1, 1 - b)
        _wait_gather(b)
        if not first:
            _wait_scatter(b)
        _compute(b)
        pltpu.async_copy(wbuf[b], s_sh.at[dsts.at[ci]], sems[b], add=True)

    _issue(0, 0)
    _half(0, 0, True)
    _half(1, 1, True)

    @pl.loop(1, NCHUNK // 2)
    def _pipe(i2):
        _half(2 * i2, 0, False)
        _half(2 * i2 + 1, 1, False)

    _wait_scatter(0)
    _wait_scatter(1)
    plsc.subcore_barrier()

    @pl.when(sid == 0)
    def _out():
        pltpu.sync_copy(s_sh, s2_hbm.at[cid])


_edge_kernel_cache = []


def _get_edge_kernel():
    if not _edge_kernel_cache:
        _edge_kernel_cache.append(_make_edge_kernel())
    return _edge_kernel_cache[0]


def _make_edge_kernel():
  return functools.partial(
    pl.kernel,
    out_type=jax.ShapeDtypeStruct((2, NPAD, 16), jnp.float32),
    mesh=plsc.VectorSubcoreMesh(core_axis_name="c", subcore_axis_name="s",
                                num_cores=2, num_subcores=16),
    compiler_params=pltpu.CompilerParams(needs_layout_passes=False,
                                         use_tc_tiling_on_sc=False),
    scratch_types=[
        pltpu.VMEM((NCHUNK, SCB), jnp.int32),
        pltpu.VMEM((NCHUNK, SCB), jnp.int32),
        pltpu.VMEM((SCB, H), jnp.bfloat16),
        pltpu.VMEM((SCB, H), jnp.bfloat16),
        pltpu.VMEM((SCB, H), jnp.bfloat16),
        pltpu.VMEM((SCB, H), jnp.bfloat16),
        pltpu.VMEM((SCB, 16), jnp.float32),
        pltpu.VMEM((SCB, 16), jnp.float32),
        pltpu.VMEM((SCB, 16), jnp.float32),
        pltpu.VMEM((SCB, 16), jnp.float32),
        pltpu.VMEM((SCB, 16), jnp.float32),
        pltpu.SemaphoreType.DMA,
        pltpu.SemaphoreType.DMA,
        pltpu.SemaphoreType.DMA,
        pltpu.SemaphoreType.DMA,
        pltpu.SemaphoreType.DMA,
        pltpu.SemaphoreType.DMA,
        pltpu.SemaphoreType.DMA,
        pltpu.SemaphoreType.DMA,
        pltpu.SemaphoreType.DMA,
        pltpu.VMEM_SHARED((NPAD, 16), jnp.float32),
        pltpu.VMEM_SHARED((NPAD, H), jnp.bfloat16),
    ],
  )(_edge_body)


def _reduce_body(s2_ref, p_ref):
    s = s2_ref[0] + s2_ref[1]
    rows = lax.broadcasted_iota(jnp.int32, (NPAD, 1), 0)
    sv = jnp.where(rows < N, s, 0.0)
    denom = jnp.sum(sv, axis=0, keepdims=True)
    p_ref[...] = s * (1.0 / jnp.maximum(denom, 1e-30))


def _reduce(s2):
    return pl.pallas_call(
        _reduce_body,
        in_specs=[pl.BlockSpec((2, NPAD, 16), lambda: (0, 0, 0))],
        out_specs=pl.BlockSpec((NPAD, 16), lambda: (0, 0)),
        out_shape=jax.ShapeDtypeStruct((NPAD, 16), jnp.float32),
    )(s2)


def _update_body(h_ref, v_ref, p_ref, rmat, ng, nb, fw1, fb1, fw2, fb2,
                 out_ref):
    p8 = p_ref[...][:, :NH]
    h1 = h_ref[...] + (p8 @ rmat[...]) * v_ref[...]
    hn = _ln_rows(h1, ng[...], nb[...])
    out_ref[...] = h1 + (_gelu(hn @ fw1[...] + fb1[...]) @ fw2[...] + fb2[...])


def _update(h, v1, p, rmat, ng, nb, fp):
    nspec = pl.BlockSpec((BN, H), lambda i: (i, 0))
    return pl.pallas_call(
        _update_body,
        grid=(NBLK,),
        in_specs=[nspec, nspec,
                  pl.BlockSpec((BN, 16), lambda i: (i, 0)),
                  pl.BlockSpec((NH, H), lambda i: (0, 0)),
                  pl.BlockSpec((1, H), lambda i: (0, 0)),
                  pl.BlockSpec((1, H), lambda i: (0, 0)),
                  pl.BlockSpec((H, 2 * H), lambda i: (0, 0)),
                  pl.BlockSpec((1, 2 * H), lambda i: (0, 0)),
                  pl.BlockSpec((2 * H, H), lambda i: (0, 0)),
                  pl.BlockSpec((1, H), lambda i: (0, 0))],
        out_specs=nspec,
        out_shape=jax.ShapeDtypeStruct((NPAD, H), jnp.float32),
    )(h, v1, p, rmat, ng, nb, fp["W1"], fp["b1"].reshape(1, -1),
      fp["W2"], fp["b2"].reshape(1, -1))


def _pool_body(h_ref, b_ref, f1w, f1b, f2w, f2b, out_ref, sums_s, cnt_s):
    i = pl.program_id(0)

    @pl.when(i == 0)
    def _():
        sums_s[...] = jnp.zeros_like(sums_s)
        cnt_s[...] = jnp.zeros_like(cnt_s)

    hb = h_ref[...]
    bv = b_ref[0]
    gi = lax.broadcasted_iota(jnp.int32, (G, BN), 0)
    cols = i * BN + lax.broadcasted_iota(jnp.int32, (G, BN), 1)
    oh = jnp.where((jnp.broadcast_to(bv, (G, BN)) == gi) & (cols < N),
                   1.0, 0.0)
    sums_s[...] += oh @ hb
    cnt_s[...] += jnp.sum(oh, axis=1, keepdims=True)

    @pl.when(i == NBLK - 1)
    def _():
        pooled = sums_s[...] / jnp.maximum(cnt_s[...], 1.0)
        hid = _gelu(pooled @ f1w[...] + f1b[...])
        out_ref[...] = hid @ f2w[...] + f2b[...]


def _pool(h, b3, f1w, f1b, f2w, f2b):
    return pl.pallas_call(
        _pool_body,
        grid=(NBLK,),
        in_specs=[pl.BlockSpec((BN, H), lambda i: (i, 0)),
                  pl.BlockSpec((1, 1, BN), lambda i: (i, 0, 0)),
                  pl.BlockSpec((H, 2 * H), lambda i: (0, 0)),
                  pl.BlockSpec((1, 2 * H), lambda i: (0, 0)),
                  pl.BlockSpec((2 * H, 1), lambda i: (0, 0)),
                  pl.BlockSpec((1, 1), lambda i: (0, 0))],
        out_specs=pl.BlockSpec((G, 1), lambda i: (0, 0)),
        out_shape=jax.ShapeDtypeStruct((G, 1), jnp.float32),
        scratch_shapes=[pltpu.VMEM((G, H), jnp.float32),
                        pltpu.VMEM((G, 1), jnp.float32)],
    )(h, b3, f1w, f1b, f2w, f2b)


def kernel(x, edge_index, edge_attr, batch, params):
    f32 = jnp.float32
    scale = jnp.sqrt(jnp.float32(DH))

    xp = jnp.zeros((NPAD, 16), f32).at[:N, :15].set(x)
    srcp = (jnp.zeros((EPAD,), jnp.int32).at[:E].set(edge_index[0])
            .reshape(NW, NCHUNK, SCB))
    dstp = (jnp.full((EPAD,), N, jnp.int32).at[:E].set(edge_index[1])
            .reshape(NW, NCHUNK, SCB))
    eap = jnp.zeros((EPAD, 8), f32).at[:E, :4].set(edge_attr[:, 3:7])
    b3 = jnp.zeros((NPAD,), jnp.int32).at[:N].set(batch).reshape(NBLK, 1, BN)

    pw = jnp.zeros((16, H), f32).at[:5].set(params["prot_W"])
    lw = jnp.zeros((16, H), f32).at[5:15].set(params["lig_W"])
    row = lambda a: a.reshape(1, -1)

    layers = params["layers"]
    w1c = jnp.zeros((8, 256), f32)
    b1c = jnp.zeros((1, 256), f32)
    w2c = jnp.zeros((256, 16), f32)
    b2c = jnp.zeros((1, 64), f32)
    for l, p in enumerate(layers):
        w16 = p["attn_W"][2 * DH:, 0]
        w2h = jnp.sum(p["sp"]["W2"].reshape(-1, NH, DH) * w16, -1)
        b2h = jnp.sum(p["sp"]["b2"].reshape(NH, DH) * w16, -1)
        w1c = w1c.at[:4, l * 64:(l + 1) * 64].set(p["sp"]["W1"])
        b1c = b1c.at[0, l * 64:(l + 1) * 64].set(p["sp"]["b1"])
        w2c = w2c.at[l * 64:(l + 1) * 64, :NH].set(w2h)
        b2c = b2c.at[0, l * 16:l * 16 + NH].set(b2h + p["attn_b"][0])

    rmat = jnp.repeat(jnp.eye(NH, dtype=f32), DH, axis=1)

    h = _embed(xp, pw, row(params["prot_b"]), row(params["prot_g"]),
               row(params["prot_beta"]), lw, row(params["lig_b"]),
               row(params["lig_g"]), row(params["lig_beta"]))

    spc = _spc(eap, w1c, b1c, w2c, b2c)

    for l, p in enumerate(layers):
        cs = jnp.full((1, H), jnp.sum(p["attn_W"][:2 * DH, 0]) / scale, f32)
        q1, k1s, v1 = _qkv(h, row(p["norm_g"]), row(p["norm_b"]),
                           p["q"], p["k"], p["v"], cs)
        s2 = _get_edge_kernel()(q1, k1s, spc[l], srcp, dstp)
        pmat = _reduce(s2)
        h = _update(h, v1, pmat, rmat, row(p["norm_g"]), row(p["norm_b"]),
                    p["ffn"])

    out = _pool(h, b3, params["fc1_W"], row(params["fc1_b"]),
                params["fc2_W"], row(params["fc2_b"]))
    return out.reshape(G)


# speedup vs baseline: 1.0164x; 1.0164x over previous
import functools

import jax
import jax.numpy as jnp
from jax import lax
from jax.experimental import pallas as pl
from jax.experimental.pallas import tpu as pltpu
from jax.experimental.pallas import tpu_sc as plsc

N = 10000
E = 160000
G = 16
H = 128
NH = 8
DH = 16

NPAD = 10240
BN = 2048
NBLK = NPAD // BN
EPAD = 163840
EB = 2048
NEBLK = EPAD // EB
NW = 32
EPW = EPAD // NW
SCB = 128
NCHUNK = EPW // SCB
NGRP = SCB // 16
NSUB = 8
SUBR = SCB // NSUB
ZROWS = NPAD // 16


def _gelu(x):
    return 0.5 * x * (1.0 + lax.erf(x * 0.7071067811865476))


def _ln_rows(x, g, b):
    m = jnp.mean(x, axis=-1, keepdims=True)
    v = jnp.mean((x - m) ** 2, axis=-1, keepdims=True)
    return (x - m) * jax.lax.rsqrt(v + 1e-5) * g + b


def _embed_body(x_ref, pw, pb, pg, pbe, lw, lb, lg, lbe, out_ref):
    i = pl.program_id(0)
    xb = x_ref[...]
    rows = i * BN + lax.broadcasted_iota(jnp.int32, (BN, 1), 0)
    is_prot = rows < 512
    lin_p = xb @ pw[...] + pb[...]
    lin_l = xb @ lw[...] + lb[...]
    lin = jnp.where(is_prot, lin_p, lin_l)
    g = jnp.where(is_prot, pg[...], lg[...])
    b = jnp.where(is_prot, pbe[...], lbe[...])
    out_ref[...] = _gelu(_ln_rows(lin, g, b))


def _embed(xp, pw, pb, pg, pbe, lw, lb, lg, lbe):
    full = pl.BlockSpec((1, H), lambda i: (0, 0))
    wspec = pl.BlockSpec((16, H), lambda i: (0, 0))
    return pl.pallas_call(
        _embed_body,
        grid=(NBLK,),
        in_specs=[pl.BlockSpec((BN, 16), lambda i: (i, 0)),
                  wspec, full, full, full, wspec, full, full, full],
        out_specs=pl.BlockSpec((BN, H), lambda i: (i, 0)),
        out_shape=jax.ShapeDtypeStruct((NPAD, H), jnp.float32),
    )(xp, pw, pb, pg, pbe, lw, lb, lg, lbe)


def _qkv_body(h_ref, ng, nb, qw1, qb1, qw2, qb2, kw1, kb1, kw2, kb2,
              vw1, vb1, vw2, vb2, cs, q_ref, k_ref, v_ref):
    hn = _ln_rows(h_ref[...], ng[...], nb[...])
    q = _gelu(hn @ qw1[...] + qb1[...]) @ qw2[...] + qb2[...]
    q_ref[...] = q.astype(jnp.bfloat16)
    k = (_gelu(hn @ kw1[...] + kb1[...]) @ kw2[...] + kb2[...]) * cs[...]
    k_ref[...] = k.astype(jnp.bfloat16)
    v_ref[...] = _gelu(hn @ vw1[...] + vb1[...]) @ vw2[...] + vb2[...]


def _qkv(h, ng, nb, qp, kp, vp, cs):
    b1 = pl.BlockSpec((1, 2 * H), lambda i: (0, 0))
    b2 = pl.BlockSpec((1, H), lambda i: (0, 0))
    w1 = pl.BlockSpec((H, 2 * H), lambda i: (0, 0))
    w2 = pl.BlockSpec((2 * H, H), lambda i: (0, 0))
    nspec = pl.BlockSpec((BN, H), lambda i: (i, 0))
    return pl.pallas_call(
        _qkv_body,
        grid=(NBLK,),
        in_specs=[nspec, b2, b2, w1, b1, w2, b2, w1, b1, w2, b2,
                  w1, b1, w2, b2, b2],
        out_specs=[nspec, nspec, nspec],
        out_shape=[jax.ShapeDtypeStruct((NPAD, H), jnp.bfloat16),
                   jax.ShapeDtypeStruct((NPAD, H), jnp.bfloat16),
                   jax.ShapeDtypeStruct((NPAD, H), jnp.float32)],
    )(h, ng, nb,
      qp["W1"], qp["b1"].reshape(1, -1), qp["W2"], qp["b2"].reshape(1, -1),
      kp["W1"], kp["b1"].reshape(1, -1), kp["W2"], kp["b2"].reshape(1, -1),
      vp["W1"], vp["b1"].reshape(1, -1), vp["W2"], vp["b2"].reshape(1, -1),
      cs)


def _spc_body(ea_ref, w1, b1, w2, b2, o0, o1, o2, o3):
    hid = jax.nn.relu(ea_ref[...] @ w1[...] + b1[...])
    for l, o in enumerate((o0, o1, o2, o3)):
        wl = w2[pl.ds(l * 64, 64), :]
        bl = b2[:, pl.ds(l * 16, 16)]
        o[...] = hid[:, l * 64:(l + 1) * 64] @ wl + bl


def _spc(eap, w1, b1, w2, b2):
    espec = pl.BlockSpec((EB, 8), lambda i: (i, 0))
    ospec = pl.BlockSpec((EB, 16), lambda i: (i, 0))
    return pl.pallas_call(
        _spc_body,
        grid=(NEBLK,),
        in_specs=[espec,
                  pl.BlockSpec((8, 256), lambda i: (0, 0)),
                  pl.BlockSpec((1, 256), lambda i: (0, 0)),
                  pl.BlockSpec((256, 16), lambda i: (0, 0)),
                  pl.BlockSpec((1, 64), lambda i: (0, 0))],
        out_specs=[ospec] * 4,
        out_shape=[jax.ShapeDtypeStruct((EPAD, 16), jnp.float32)] * 4,
    )(eap, w1, b1, w2, b2)


def _edge_body(q_hbm, k_hbm, spc_hbm, src_hbm, dst_hbm, s2_hbm,
               srcs, dsts, qrows0, qrows1, krows0, krows1, spcv0, spcv1,
               wbuf0, wbuf1, zbuf,
               semq0, semq1, semk0, semk1, semp0, semp1, sems0, sems1,
               s_sh):
    cid = lax.axis_index("c")
    sid = lax.axis_index("s")
    wid = cid * 16 + sid
    ebase = wid * EPW

    qrows = (qrows0, qrows1)
    krows = (krows0, krows1)
    spcv = (spcv0, spcv1)
    wbuf = (wbuf0, wbuf1)
    semq = (semq0, semq1)
    semk = (semk0, semk1)
    semp = (semp0, semp1)
    sems = (sems0, sems1)

    pltpu.sync_copy(src_hbm.at[wid], srcs)
    pltpu.sync_copy(dst_hbm.at[wid], dsts)

    @pl.loop(0, SCB)
    def _zw(r):
        z = jnp.zeros((16,), jnp.float32)
        wbuf0[r, :] = z
        wbuf1[r, :] = z
        zbuf[r, :] = z

    for j in range(ZROWS // SCB):
        pltpu.sync_copy(zbuf, s_sh.at[pl.ds(sid * ZROWS + j * SCB, SCB), :])
    plsc.subcore_barrier()

    lane = lax.broadcasted_iota(jnp.int32, (16,), 0)

    def _issue(ci, b):
        goff = pl.multiple_of(ebase + ci * SCB, 8)
        for s in range(NSUB):
            rs = pl.ds(s * SUBR, SUBR)
            pltpu.async_copy(q_hbm.at[srcs.at[ci, rs]],
                             qrows[b].at[rs, :], semq[b])
            pltpu.async_copy(k_hbm.at[dsts.at[ci, rs]],
                             krows[b].at[rs, :], semk[b])
        pltpu.async_copy(spc_hbm.at[pl.ds(goff, SCB), :], spcv[b], semp[b])

    def _wait_gather(b):
        for s in range(NSUB):
            rs = pl.ds(s * SUBR, SUBR)
            pltpu.make_async_copy(q_hbm.at[pl.ds(0, SUBR), :],
                                  qrows[b].at[rs, :], semq[b]).wait()
            pltpu.make_async_copy(k_hbm.at[pl.ds(0, SUBR), :],
                                  krows[b].at[rs, :], semk[b]).wait()
        pltpu.make_async_copy(spc_hbm.at[pl.ds(0, SCB), :], spcv[b], semp[b]).wait()

    def _wait_scatter(b):
        pltpu.make_async_copy(wbuf[b], s_sh.at[pl.ds(0, SCB), :], sems[b]).wait()

    lane7 = lane == 7
    lane15 = lane == 15
    lane_hi = lane >= 8

    def _compute(b):
        qb, kb, pb, wb = qrows[b], krows[b], spcv[b], wbuf[b]

        @pl.loop(0, SCB)
        def _edge(e):
            erep = jnp.full((16,), e, jnp.int32)
            for j in range(NH // 2):
                qp = qb[e, pl.ds(j * 2 * DH, 2 * DH)]
                kp = kb[e, pl.ds(j * 2 * DH, 2 * DH)]
                qa, qo = plsc.unpack(qp, format=plsc.PackFormat.INTERLEAVED)
                ka, ko = plsc.unpack(kp, format=plsc.PackFormat.INTERLEAVED)
                prod = qa * ka + qo * ko
                s_lo = plsc.cumsum(prod)
                s_hi = plsc.cumsum(jnp.where(lane_hi, prod, 0.0))
                plsc.store_scatter(
                    wb, [erep, jnp.full((16,), 2 * j, jnp.int32)], s_lo,
                    mask=lane7)
                plsc.store_scatter(
                    wb, [erep, jnp.full((16,), 2 * j + 1, jnp.int32)], s_hi,
                    mask=lane15)

        @pl.loop(0, SCB)
        def _edge2(e):
            logit = wb[e, :] + pb[e, :]
            sig = 1.0 / (1.0 + jnp.exp(-logit))
            wb[e, :] = jnp.exp(sig)

    def _half(ci, b, first):
        if isinstance(ci, int):
            if ci + 1 < NCHUNK:
                _issue(ci + 1, 1 - b)
        else:
            @pl.when(ci + 1 < NCHUNK)
            def _():
                _issue(ci + 1, 1 - b)
        _wait_gather(b)
        if not first:
            _wait_scatter(b)
        _compute(b)
        pltpu.async_copy(wbuf[b], s_sh.at[dsts.at[ci]], sems[b], add=True)

    _issue(0, 0)
    _half(0, 0, True)
    _half(1, 1, True)

    @pl.loop(1, NCHUNK // 2)
    def _pipe(i2):
        _half(2 * i2, 0, False)
        _half(2 * i2 + 1, 1, False)

    _wait_scatter(0)
    _wait_scatter(1)
    plsc.subcore_barrier()

    acc0 = jnp.zeros((16,), jnp.float32)
    for j in range(ZROWS // SCB):
        pltpu.sync_copy(s_sh.at[pl.ds(sid * ZROWS + j * SCB, SCB), :], zbuf)

        def _sum_rows(r, acc):
            grow = sid * ZROWS + j * SCB + r
            return acc + jnp.where(grow < N, zbuf[r, :], 0.0)

        acc0 = lax.fori_loop(0, SCB, _sum_rows, acc0)
    wbuf0[0, :] = acc0
    pltpu.sync_copy(wbuf0.at[pl.ds(0, 1), :],
                    s2_hbm.at[cid, pl.ds(NPAD + sid, 1), :])

    @pl.when(sid == 0)
    def _out():
        pltpu.sync_copy(s_sh, s2_hbm.at[cid, pl.ds(0, NPAD), :])


_edge_kernel_cache = []


def _get_edge_kernel():
    if not _edge_kernel_cache:
        _edge_kernel_cache.append(_make_edge_kernel())
    return _edge_kernel_cache[0]


def _make_edge_kernel():
  return functools.partial(
    pl.kernel,
    out_type=jax.ShapeDtypeStruct((2, NPAD + 16, 16), jnp.float32),
    mesh=plsc.VectorSubcoreMesh(core_axis_name="c", subcore_axis_name="s",
                                num_cores=2, num_subcores=16),
    compiler_params=pltpu.CompilerParams(needs_layout_passes=False,
                                         use_tc_tiling_on_sc=False),
    scratch_types=[
        pltpu.VMEM((NCHUNK, SCB), jnp.int32),
        pltpu.VMEM((NCHUNK, SCB), jnp.int32),
        pltpu.VMEM((SCB, H), jnp.bfloat16),
        pltpu.VMEM((SCB, H), jnp.bfloat16),
        pltpu.VMEM((SCB, H), jnp.bfloat16),
        pltpu.VMEM((SCB, H), jnp.bfloat16),
        pltpu.VMEM((SCB, 16), jnp.float32),
        pltpu.VMEM((SCB, 16), jnp.float32),
        pltpu.VMEM((SCB, 16), jnp.float32),
        pltpu.VMEM((SCB, 16), jnp.float32),
        pltpu.VMEM((SCB, 16), jnp.float32),
        pltpu.SemaphoreType.DMA,
        pltpu.SemaphoreType.DMA,
        pltpu.SemaphoreType.DMA,
        pltpu.SemaphoreType.DMA,
        pltpu.SemaphoreType.DMA,
        pltpu.SemaphoreType.DMA,
        pltpu.SemaphoreType.DMA,
        pltpu.SemaphoreType.DMA,
        pltpu.VMEM_SHARED((NPAD, 16), jnp.float32),
    ],
  )(_edge_body)


def _agg_ffn(h_ref, v_ref, s2_ref, d_ref, rmat, ng, nb, fw1, fb1, fw2, fb2):
    dsum = d_ref[0] + d_ref[1]
    denom = jnp.sum(dsum, axis=0, keepdims=True)
    inv = 1.0 / jnp.maximum(denom, 1e-30)
    p8 = ((s2_ref[0] + s2_ref[1]) * inv)[:, :NH]
    h1 = h_ref[...] + (p8 @ rmat[...]) * v_ref[...]
    hn = _ln_rows(h1, ng[...], nb[...])
    return h1 + (_gelu(hn @ fw1[...] + fb1[...]) @ fw2[...] + fb2[...])


def _update_body(h_ref, v_ref, s2_ref, d_ref, rmat, ng, nb,
                 fw1, fb1, fw2, fb2, out_ref):
    out_ref[...] = _agg_ffn(h_ref, v_ref, s2_ref, d_ref, rmat, ng, nb,
                            fw1, fb1, fw2, fb2)


def _updqkv_body(h_ref, v_ref, s2_ref, d_ref, rmat, ng, nb,
                 fw1, fb1, fw2, fb2, ng2, nb2,
                 qw1, qb1, qw2, qb2, kw1, kb1, kw2, kb2,
                 vw1, vb1, vw2, vb2, cs,
                 out_ref, q_ref, k_ref, vo_ref):
    h2 = _agg_ffn(h_ref, v_ref, s2_ref, d_ref, rmat, ng, nb,
                  fw1, fb1, fw2, fb2)
    out_ref[...] = h2
    hn2 = _ln_rows(h2, ng2[...], nb2[...])
    q = _gelu(hn2 @ qw1[...] + qb1[...]) @ qw2[...] + qb2[...]
    q_ref[...] = q.astype(jnp.bfloat16)
    k = (_gelu(hn2 @ kw1[...] + kb1[...]) @ kw2[...] + kb2[...]) * cs[...]
    k_ref[...] = k.astype(jnp.bfloat16)
    vo_ref[...] = _gelu(hn2 @ vw1[...] + vb1[...]) @ vw2[...] + vb2[...]


def _upd_specs():
    nspec = pl.BlockSpec((BN, H), lambda i: (i, 0))
    b1 = pl.BlockSpec((1, 2 * H), lambda i: (0, 0))
    b2 = pl.BlockSpec((1, H), lambda i: (0, 0))
    w1 = pl.BlockSpec((H, 2 * H), lambda i: (0, 0))
    w2 = pl.BlockSpec((2 * H, H), lambda i: (0, 0))
    base = [nspec, nspec,
            pl.BlockSpec((2, BN, 16), lambda i: (0, i, 0)),
            pl.BlockSpec((2, 16, 16), lambda i: (0, NPAD // 16, 0)),
            pl.BlockSpec((NH, H), lambda i: (0, 0)),
            b2, b2, w1, b1, w2, b2]
    return nspec, b1, b2, w1, w2, base


def _update(h, v1, s2, rmat, ng, nb, fp):
    nspec, b1, b2, w1, w2, base = _upd_specs()
    return pl.pallas_call(
        _update_body,
        grid=(NBLK,),
        in_specs=base,
        out_specs=nspec,
        out_shape=jax.ShapeDtypeStruct((NPAD, H), jnp.float32),
    )(h, v1, s2, s2, rmat, ng, nb, fp["W1"], fp["b1"].reshape(1, -1),
      fp["W2"], fp["b2"].reshape(1, -1))


def _updqkv(h, v1, s2, rmat, ng, nb, fp, ng2, nb2, qp, kp, vp, cs):
    nspec, b1, b2, w1, w2, base = _upd_specs()
    return pl.pallas_call(
        _updqkv_body,
        grid=(NBLK,),
        in_specs=base + [b2, b2, w1, b1, w2, b2, w1, b1, w2, b2,
                         w1, b1, w2, b2, b2],
        out_specs=[nspec] * 4,
        out_shape=[jax.ShapeDtypeStruct((NPAD, H), jnp.float32),
                   jax.ShapeDtypeStruct((NPAD, H), jnp.bfloat16),
                   jax.ShapeDtypeStruct((NPAD, H), jnp.bfloat16),
                   jax.ShapeDtypeStruct((NPAD, H), jnp.float32)],
    )(h, v1, s2, s2, rmat, ng, nb, fp["W1"], fp["b1"].reshape(1, -1),
      fp["W2"], fp["b2"].reshape(1, -1), ng2, nb2,
      qp["W1"], qp["b1"].reshape(1, -1), qp["W2"], qp["b2"].reshape(1, -1),
      kp["W1"], kp["b1"].reshape(1, -1), kp["W2"], kp["b2"].reshape(1, -1),
      vp["W1"], vp["b1"].reshape(1, -1), vp["W2"], vp["b2"].reshape(1, -1),
      cs)


def _pool_body(h_ref, b_ref, f1w, f1b, f2w, f2b, out_ref, sums_s, cnt_s):
    i = pl.program_id(0)

    @pl.when(i == 0)
    def _():
        sums_s[...] = jnp.zeros_like(sums_s)
        cnt_s[...] = jnp.zeros_like(cnt_s)

    hb = h_ref[...]
    bv = b_ref[0]
    gi = lax.broadcasted_iota(jnp.int32, (G, BN), 0)
    cols = i * BN + lax.broadcasted_iota(jnp.int32, (G, BN), 1)
    oh = jnp.where((jnp.broadcast_to(bv, (G, BN)) == gi) & (cols < N),
                   1.0, 0.0)
    sums_s[...] += oh @ hb
    cnt_s[...] += jnp.sum(oh, axis=1, keepdims=True)

    @pl.when(i == NBLK - 1)
    def _():
        pooled = sums_s[...] / jnp.maximum(cnt_s[...], 1.0)
        hid = _gelu(pooled @ f1w[...] + f1b[...])
        out_ref[...] = hid @ f2w[...] + f2b[...]


def _pool(h, b3, f1w, f1b, f2w, f2b):
    return pl.pallas_call(
        _pool_body,
        grid=(NBLK,),
        in_specs=[pl.BlockSpec((BN, H), lambda i: (i, 0)),
                  pl.BlockSpec((1, 1, BN), lambda i: (i, 0, 0)),
                  pl.BlockSpec((H, 2 * H), lambda i: (0, 0)),
                  pl.BlockSpec((1, 2 * H), lambda i: (0, 0)),
                  pl.BlockSpec((2 * H, 1), lambda i: (0, 0)),
                  pl.BlockSpec((1, 1), lambda i: (0, 0))],
        out_specs=pl.BlockSpec((G, 1), lambda i: (0, 0)),
        out_shape=jax.ShapeDtypeStruct((G, 1), jnp.float32),
        scratch_shapes=[pltpu.VMEM((G, H), jnp.float32),
                        pltpu.VMEM((G, 1), jnp.float32)],
    )(h, b3, f1w, f1b, f2w, f2b)


def kernel(x, edge_index, edge_attr, batch, params):
    f32 = jnp.float32
    scale = jnp.sqrt(jnp.float32(DH))

    xp = jnp.zeros((NPAD, 16), f32).at[:N, :15].set(x)
    srcp = (jnp.zeros((EPAD,), jnp.int32).at[:E].set(edge_index[0])
            .reshape(NW, NCHUNK, SCB))
    dstp = (jnp.full((EPAD,), N, jnp.int32).at[:E].set(edge_index[1])
            .reshape(NW, NCHUNK, SCB))
    eap = jnp.zeros((EPAD, 8), f32).at[:E, :4].set(edge_attr[:, 3:7])
    b3 = jnp.zeros((NPAD,), jnp.int32).at[:N].set(batch).reshape(NBLK, 1, BN)

    pw = jnp.zeros((16, H), f32).at[:5].set(params["prot_W"])
    lw = jnp.zeros((16, H), f32).at[5:15].set(params["lig_W"])
    row = lambda a: a.reshape(1, -1)

    layers = params["layers"]
    w1c = jnp.zeros((8, 256), f32)
    b1c = jnp.zeros((1, 256), f32)
    w2c = jnp.zeros((256, 16), f32)
    b2c = jnp.zeros((1, 64), f32)
    for l, p in enumerate(layers):
        w16 = p["attn_W"][2 * DH:, 0]
        w2h = jnp.sum(p["sp"]["W2"].reshape(-1, NH, DH) * w16, -1)
        b2h = jnp.sum(p["sp"]["b2"].reshape(NH, DH) * w16, -1)
        w1c = w1c.at[:4, l * 64:(l + 1) * 64].set(p["sp"]["W1"])
        b1c = b1c.at[0, l * 64:(l + 1) * 64].set(p["sp"]["b1"])
        w2c = w2c.at[l * 64:(l + 1) * 64, :NH].set(w2h)
        b2c = b2c.at[0, l * 16:l * 16 + NH].set(b2h + p["attn_b"][0])

    rmat = jnp.repeat(jnp.eye(NH, dtype=f32), DH, axis=1)

    h = _embed(xp, pw, row(params["prot_b"]), row(params["prot_g"]),
               row(params["prot_beta"]), lw, row(params["lig_b"]),
               row(params["lig_g"]), row(params["lig_beta"]))

    spc = _spc(eap, w1c, b1c, w2c, b2c)

    def _cs(p):
        return jnp.full((1, H), jnp.sum(p["attn_W"][:2 * DH, 0]) / scale, f32)

    p0 = layers[0]
    qbf, kbf, v1 = _qkv(h, row(p0["norm_g"]), row(p0["norm_b"]),
                        p0["q"], p0["k"], p0["v"], _cs(p0))
    for l, p in enumerate(layers):
        s2 = _get_edge_kernel()(qbf, kbf, spc[l], srcp, dstp)
        if l + 1 < len(layers):
            pn = layers[l + 1]
            h, qbf, kbf, v1 = _updqkv(
                h, v1, s2, rmat, row(p["norm_g"]), row(p["norm_b"]), p["ffn"],
                row(pn["norm_g"]), row(pn["norm_b"]),
                pn["q"], pn["k"], pn["v"], _cs(pn))
        else:
            h = _update(h, v1, s2, rmat, row(p["norm_g"]), row(p["norm_b"]),
                        p["ffn"])

    out = _pool(h, b3, params["fc1_W"], row(params["fc1_b"]),
                params["fc2_W"], row(params["fc2_b"]))
    return out.reshape(G)
